# baseline (speedup 1.0000x reference)
import functools

import jax
import jax.numpy as jnp
from jax import lax
from jax.experimental import pallas as pl
from jax.experimental.pallas import tpu as pltpu
from jax.experimental.pallas import tpu_sc as plsc

_EPS = 1e-15
_B = 16384
_D = 64
_WIN = 20
_NC = 2
_NS = 16
_NW = _NC * _NS
_BPW = _B // _NW
_BLK = 128
_NBLK = _BPW // _BLK


def _sc_body(pos_hbm, neg_hbm, ctxT_hbm, tw_hbm, cw_hbm, opos_hbm, oneg_hbm,
             idx_pos, idx_neg, idx_ctx, pos_rows, neg_rows, acc,
             stage_pos, stage_neg, sem_idx, sem_g, sem_a, sem_o):
    wid = lax.axis_index("s") * _NC + lax.axis_index("c")
    base = wid * _BPW

    idx_cps = []
    for j in range(_NBLK):
        off = base + j * _BLK
        idx_cps.append(pltpu.async_copy(
            pos_hbm.at[pl.ds(off, _BLK)], idx_pos.at[j], sem_idx))
        idx_cps.append(pltpu.async_copy(
            neg_hbm.at[pl.ds(off, _BLK)], idx_neg.at[j], sem_idx))
        for w in range(_WIN):
            idx_cps.append(pltpu.async_copy(
                ctxT_hbm.at[w, pl.ds(off, _BLK)], idx_ctx.at[w, j], sem_idx))
    for c in idx_cps:
        c.wait()

    g1 = []
    for j in range(_NBLK):
        dst = pl.ds(j * _BLK, _BLK)
        g1.append(pltpu.async_copy(
            tw_hbm.at[idx_pos.at[j]], pos_rows.at[dst], sem_g))
        g1.append(pltpu.async_copy(
            tw_hbm.at[idx_neg.at[j]], neg_rows.at[dst], sem_g))
        g1.append(pltpu.async_copy(
            cw_hbm.at[idx_ctx.at[0, j]], acc.at[dst], sem_g))
    for c in g1:
        c.wait()

    g2 = []
    for j in range(_NBLK):
        dst = pl.ds(j * _BLK, _BLK)
        for w in range(1, _WIN):
            g2.append(pltpu.async_copy(
                cw_hbm.at[idx_ctx.at[w, j]], acc.at[dst], sem_a, add=True))
    for c in g2:
        c.wait()

    def elem(e, carry):
        pv = None
        nv = None
        for k in range(_D // 16):
            sl = pl.ds(k * 16, 16)
            a = acc[e, sl]
            p = pos_rows[e, sl] * a
            n = neg_rows[e, sl] * a
            pv = p if pv is None else pv + p
            nv = n if nv is None else nv + n
        stage_pos[e] = pv
        stage_neg[e] = nv
        return carry

    lax.fori_loop(0, _BPW, elem, None)

    o1 = pltpu.async_copy(stage_pos, opos_hbm.at[pl.ds(base, _BPW)], sem_o)
    o2 = pltpu.async_copy(stage_neg, oneg_hbm.at[pl.ds(base, _BPW)], sem_o)
    o1.wait()
    o2.wait()


_sc_cbow = functools.partial(
    pl.kernel,
    out_type=(jax.ShapeDtypeStruct((_B, 16), jnp.float32),
              jax.ShapeDtypeStruct((_B, 16), jnp.float32)),
    mesh=plsc.VectorSubcoreMesh(core_axis_name="c", subcore_axis_name="s",
                                num_cores=_NC, num_subcores=_NS),
    scratch_types=[
        pltpu.VMEM((_NBLK, _BLK), jnp.int32),
        pltpu.VMEM((_NBLK, _BLK), jnp.int32),
        pltpu.VMEM((_WIN, _NBLK, _BLK), jnp.int32),
        pltpu.VMEM((_BPW, _D), jnp.float32),
        pltpu.VMEM((_BPW, _D), jnp.float32),
        pltpu.VMEM((_BPW, _D), jnp.float32),
        pltpu.VMEM((_BPW, 16), jnp.float32),
        pltpu.VMEM((_BPW, 16), jnp.float32),
        pltpu.SemaphoreType.DMA,
        pltpu.SemaphoreType.DMA,
        pltpu.SemaphoreType.DMA,
        pltpu.SemaphoreType.DMA,
    ],
    compiler_params=pltpu.CompilerParams(use_tc_tiling_on_sc=False),
)(_sc_body)


def _tc_finish(pp_ref, np_ref, out_ref):
    ps = jnp.sum(pp_ref[...], axis=1) * (1.0 / _WIN) + _EPS
    ns = jnp.sum(np_ref[...], axis=1) * (1.0 / _WIN) + _EPS
    pos_score = -jax.nn.log_sigmoid(ps)
    neg_score = -jax.nn.log_sigmoid(1.0 - ns)
    out_ref[0, 0] = jnp.mean(pos_score + neg_score)


def kernel(pos_nodes, neg_nodes, context_nodes, target_weight, context_weight):
    pos = pos_nodes.astype(jnp.int32)
    neg = neg_nodes.astype(jnp.int32)
    ctxT = context_nodes.astype(jnp.int32).T

    pp, nn = _sc_cbow(pos, neg, ctxT, target_weight, context_weight)

    loss = pl.pallas_call(
        _tc_finish,
        out_shape=jax.ShapeDtypeStruct((1, 1), jnp.float32),
        out_specs=pl.BlockSpec(memory_space=pltpu.SMEM),
    )(pp, nn)
    return loss[0, 0]
